# baseline (speedup 1.0000x reference)
import functools

import jax
import jax.numpy as jnp
from jax import lax
from jax.experimental import pallas as pl
from jax.experimental.pallas import tpu as pltpu
from jax.experimental.pallas import tpu_sc as plsc

L = 200
B = 4096
D = 32
CHUNK = 128
NC, NS = 2, 16
NW = NC * NS
NCHUNKS = (L * B) // CHUNK
NBUF = 5
PG = 2
KS = 2
S = 4
LS = L // S
NCH_S = NCHUNKS // S
ROWS_PER_L = (B * D) // 128

_mesh = plsc.VectorSubcoreMesh(
    core_axis_name="c", subcore_axis_name="s", num_cores=NC, num_subcores=NS
)


def _make_embed(nchunks):
    cpw = nchunks // NW

    @functools.partial(
        pl.kernel,
        out_type=jax.ShapeDtypeStruct((nchunks, CHUNK, D), jnp.float32),
        mesh=_mesh,
        scratch_types=[
            pltpu.VMEM((cpw, CHUNK), jnp.int32),
            pltpu.VMEM((NBUF, CHUNK, D), jnp.float32),
            pltpu.SemaphoreType.DMA,
            pltpu.SemaphoreType.DMA,
        ],
        compiler_params=pltpu.CompilerParams(use_tc_tiling_on_sc=False),
    )
    def embed(table_hbm, idx_hbm, out_hbm, idx_v, ring, sem_g, sem_s):
        wid = lax.axis_index("s") * NC + lax.axis_index("c")
        c0 = wid * cpw

        pltpu.sync_copy(idx_hbm.at[pl.ds(c0, cpw)], idx_v)

        def issue_gather(j):
            pltpu.async_copy(
                table_hbm.at[idx_v.at[j]], ring.at[j % NBUF], sem_g
            )

        def wait_gather():
            pltpu.make_async_copy(
                table_hbm.at[idx_v.at[0]], ring.at[0], sem_g
            ).wait()

        def issue_scatter(j):
            pltpu.async_copy(ring.at[j % NBUF], out_hbm.at[c0 + j], sem_s)

        def wait_scatter():
            pltpu.make_async_copy(ring.at[0], out_hbm.at[0], sem_s).wait()

        for j in range(PG + 1):
            issue_gather(j)

        def body(j, _):
            wait_gather()
            issue_scatter(j)

            @pl.when(j >= KS)
            def _():
                wait_scatter()

            @pl.when(j + PG + 1 < cpw)
            def _():
                issue_gather(j + PG + 1)

            return 0

        lax.fori_loop(0, cpw, body, 0)
        for _ in range(KS):
            wait_scatter()

    return embed


_embed_slice = _make_embed(NCH_S)


def _transpose_block(flat_ref, out_ref):
    xb = flat_ref[...]
    eye = jnp.eye(128, dtype=jnp.float32)
    xt = jnp.concatenate(
        [
            lax.dot_general(
                xb[j * 128:(j + 1) * 128, :],
                eye,
                (((0,), (0,)), ((), ())),
                preferred_element_type=jnp.float32,
            )
            for j in range(ROWS_PER_L // 128)
        ],
        axis=1,
    )
    y = xt.reshape(4, D, ROWS_PER_L).transpose(1, 0, 2).reshape(D, B)
    out_ref[0] = y


def _transpose_block_acc(flat_ref, acc_ref, out_ref):
    del acc_ref
    _transpose_block(flat_ref, out_ref)


def _make_to_ldb(s):
    if s == 0:
        return pl.pallas_call(
            _transpose_block,
            grid=(LS,),
            in_specs=[pl.BlockSpec((ROWS_PER_L, 128), lambda l: (l, 0))],
            out_specs=pl.BlockSpec((1, D, B), lambda l: (l, 0, 0)),
            out_shape=jax.ShapeDtypeStruct((L, D, B), jnp.float32),
        )
    return pl.pallas_call(
        _transpose_block_acc,
        grid=(LS,),
        in_specs=[
            pl.BlockSpec((ROWS_PER_L, 128), lambda l: (l, 0)),
            pl.BlockSpec(memory_space=pl.ANY),
        ],
        out_specs=pl.BlockSpec((1, D, B), lambda l, _s=s: (_s * LS + l, 0, 0)),
        out_shape=jax.ShapeDtypeStruct((L, D, B), jnp.float32),
        input_output_aliases={1: 0},
    )


_to_ldb_slices = [_make_to_ldb(s) for s in range(S)]


def kernel(x, table):
    xt = jnp.transpose(x)
    idx = (
        xt.reshape(L, 4, 32, 32)
        .transpose(0, 2, 3, 1)
        .reshape(NCHUNKS, CHUNK)
    )
    ldb = None
    for s in range(S):
        g = _embed_slice(table, idx[s * NCH_S:(s + 1) * NCH_S])
        flat = g.reshape(LS * ROWS_PER_L, 128)
        ldb = _to_ldb_slices[s](flat) if s == 0 else _to_ldb_slices[s](flat, ldb)
    return jnp.transpose(ldb, (0, 2, 1))


# speedup vs baseline: 1.0675x; 1.0675x over previous
import functools

import jax
import jax.numpy as jnp
from jax import lax
from jax.experimental import pallas as pl
from jax.experimental.pallas import tpu as pltpu
from jax.experimental.pallas import tpu_sc as plsc

L = 200
B = 4096
D = 32
CHUNK = 128
NC, NS = 2, 16
NW = NC * NS
NCHUNKS = (L * B) // CHUNK
CPL = B // CHUNK
NBUF = 5
PG = 2
KS = 2
S = 4
LS = L // S
NCH_S = NCHUNKS // S
ROWS_PER_L = (B * D) // 128

_mesh = plsc.VectorSubcoreMesh(
    core_axis_name="c", subcore_axis_name="s", num_cores=NC, num_subcores=NS
)


def _make_embed(nchunks, s0):
    cpw = nchunks // NW

    @functools.partial(
        pl.kernel,
        out_type=jax.ShapeDtypeStruct((nchunks, CHUNK, D), jnp.float32),
        mesh=_mesh,
        scratch_types=[
            pltpu.VMEM((3, B), jnp.int32),
            pltpu.VMEM((cpw, CHUNK), jnp.int32),
            pltpu.VMEM((NBUF, CHUNK, D), jnp.float32),
            pltpu.SemaphoreType.DMA,
            pltpu.SemaphoreType.DMA,
        ],
        compiler_params=pltpu.CompilerParams(
            use_tc_tiling_on_sc=False, needs_layout_passes=False
        ),
    )
    def embed(table_hbm, xt_hbm, out_hbm, xrow_v, idx_v, ring, sem_g, sem_s):
        wid = lax.axis_index("s") * NC + lax.axis_index("c")
        c0 = wid * cpw
        gc0 = s0 + c0

        lo = jnp.minimum(gc0 // CPL, L - 3)
        pltpu.sync_copy(xt_hbm.at[pl.ds(lo, 3)], xrow_v)

        vi = lax.iota(jnp.int32, 16)
        v0 = (B // 4) * (vi & 3) + (vi >> 2)

        def perm_body(j, _):
            gc = gc0 + j
            rl = jnp.full((16,), gc // CPL - lo, jnp.int32)
            base = (gc % CPL) * (CHUNK // 4)
            for t in range(CHUNK // 16):
                vals = plsc.load_gather(xrow_v, [rl, v0 + (base + 4 * t)])
                idx_v[j, pl.ds(t * 16, 16)] = vals
            return 0

        lax.fori_loop(0, cpw, perm_body, 0)

        def issue_gather(j):
            pltpu.async_copy(
                table_hbm.at[idx_v.at[j]], ring.at[j % NBUF], sem_g
            )

        def wait_gather():
            pltpu.make_async_copy(
                table_hbm.at[idx_v.at[0]], ring.at[0], sem_g
            ).wait()

        def issue_scatter(j):
            pltpu.async_copy(ring.at[j % NBUF], out_hbm.at[c0 + j], sem_s)

        def wait_scatter():
            pltpu.make_async_copy(ring.at[0], out_hbm.at[0], sem_s).wait()

        for j in range(PG + 1):
            issue_gather(j)

        def body(j, _):
            wait_gather()
            issue_scatter(j)

            @pl.when(j >= KS)
            def _():
                wait_scatter()

            @pl.when(j + PG + 1 < cpw)
            def _():
                issue_gather(j + PG + 1)

            return 0

        lax.fori_loop(0, cpw, body, 0)
        for _ in range(KS):
            wait_scatter()

    return embed


_embed_slices = [_make_embed(NCH_S, s * NCH_S) for s in range(S)]


def _transpose_block(flat_ref, out_ref):
    xt = flat_ref[...].T
    y = xt.reshape(4, D, ROWS_PER_L).transpose(1, 0, 2).reshape(D, B)
    out_ref[0] = y


def _transpose_block_acc(flat_ref, acc_ref, out_ref):
    del acc_ref
    _transpose_block(flat_ref, out_ref)


def _make_to_ldb(s):
    if s == 0:
        return pl.pallas_call(
            _transpose_block,
            grid=(LS,),
            in_specs=[pl.BlockSpec((ROWS_PER_L, 128), lambda l: (l, 0))],
            out_specs=pl.BlockSpec((1, D, B), lambda l: (l, 0, 0)),
            out_shape=jax.ShapeDtypeStruct((L, D, B), jnp.float32),
        )
    return pl.pallas_call(
        _transpose_block_acc,
        grid=(LS,),
        in_specs=[
            pl.BlockSpec((ROWS_PER_L, 128), lambda l: (l, 0)),
            pl.BlockSpec(memory_space=pl.ANY),
        ],
        out_specs=pl.BlockSpec((1, D, B), lambda l, _s=s: (_s * LS + l, 0, 0)),
        out_shape=jax.ShapeDtypeStruct((L, D, B), jnp.float32),
        input_output_aliases={1: 0},
    )


_to_ldb_slices = [_make_to_ldb(s) for s in range(S)]


def kernel(x, table):
    xt = jnp.transpose(x)
    ldb = None
    for s in range(S):
        g = _embed_slices[s](table, xt)
        flat = g.reshape(LS * ROWS_PER_L, 128)
        ldb = _to_ldb_slices[s](flat) if s == 0 else _to_ldb_slices[s](flat, ldb)
    return jnp.transpose(ldb, (0, 2, 1))


# speedup vs baseline: 1.0804x; 1.0121x over previous
import functools

import jax
import jax.numpy as jnp
from jax import lax
from jax.experimental import pallas as pl
from jax.experimental.pallas import tpu as pltpu
from jax.experimental.pallas import tpu_sc as plsc

L = 200
B = 4096
D = 32
CHUNK = 128
NC, NS = 2, 16
NW = NC * NS
NCHUNKS = (L * B) // CHUNK
CPL = B // CHUNK
NBUF = 5
PG = 2
KS = 2
S = 4
LS = L // S
NCH_S = NCHUNKS // S
ROWS_PER_L = (B * D) // 128

_mesh = plsc.VectorSubcoreMesh(
    core_axis_name="c", subcore_axis_name="s", num_cores=NC, num_subcores=NS
)


def _make_embed(nchunks, s0):
    cpw = nchunks // NW

    @functools.partial(
        pl.kernel,
        out_type=jax.ShapeDtypeStruct((nchunks, CHUNK, D), jnp.float32),
        mesh=_mesh,
        scratch_types=[
            pltpu.VMEM((3, B), jnp.int32),
            pltpu.VMEM((cpw, CHUNK), jnp.int32),
            pltpu.VMEM((NBUF, CHUNK, D), jnp.float32),
            pltpu.SemaphoreType.DMA,
            pltpu.SemaphoreType.DMA,
        ],
        compiler_params=pltpu.CompilerParams(
            use_tc_tiling_on_sc=False, needs_layout_passes=False
        ),
    )
    def embed(table_hbm, xt_hbm, out_hbm, xrow_v, idx_v, ring, sem_g, sem_s):
        wid = lax.axis_index("s") * NC + lax.axis_index("c")
        c0 = wid * cpw
        gc0 = s0 + c0

        lo = jnp.minimum(gc0 // CPL, L - 3)
        pltpu.sync_copy(xt_hbm.at[pl.ds(lo, 3)], xrow_v)

        vi = lax.iota(jnp.int32, 16)
        v0 = (B // 4) * (vi & 3) + (vi >> 2)

        def perm_body(j, _):
            gc = gc0 + j
            rl = jnp.full((16,), gc // CPL - lo, jnp.int32)
            base = (gc % CPL) * (CHUNK // 4)
            for t in range(CHUNK // 16):
                vals = plsc.load_gather(xrow_v, [rl, v0 + (base + 4 * t)])
                idx_v[j, pl.ds(t * 16, 16)] = vals
            return 0

        lax.fori_loop(0, cpw, perm_body, 0)

        def issue_gather(j):
            pltpu.async_copy(
                table_hbm.at[idx_v.at[j]], ring.at[j % NBUF], sem_g
            )

        def wait_gather():
            pltpu.make_async_copy(
                table_hbm.at[idx_v.at[0]], ring.at[0], sem_g
            ).wait()

        def issue_scatter(j):
            pltpu.async_copy(ring.at[j % NBUF], out_hbm.at[c0 + j], sem_s)

        def wait_scatter():
            pltpu.make_async_copy(ring.at[0], out_hbm.at[0], sem_s).wait()

        for j in range(PG + 1):
            issue_gather(j)

        def body(j, _):
            wait_gather()
            issue_scatter(j)

            @pl.when(j >= KS)
            def _():
                wait_scatter()

            @pl.when(j + PG + 1 < cpw)
            def _():
                issue_gather(j + PG + 1)

            return 0

        lax.fori_loop(0, cpw, body, 0)
        for _ in range(KS):
            wait_scatter()

    return embed


_embed_slices = [_make_embed(NCH_S, s * NCH_S) for s in range(S)]


def _transpose_block(flat_ref, out_ref):
    xt = flat_ref[...].T
    for h in range(4):
        out_ref[0, :, h * ROWS_PER_L:(h + 1) * ROWS_PER_L] = (
            xt[h * D:(h + 1) * D, :]
        )


def _transpose_block_acc(flat_ref, acc_ref, out_ref):
    del acc_ref
    _transpose_block(flat_ref, out_ref)


def _make_to_ldb(s):
    if s == 0:
        return pl.pallas_call(
            _transpose_block,
            grid=(LS,),
            in_specs=[pl.BlockSpec((ROWS_PER_L, 128), lambda l: (l, 0))],
            out_specs=pl.BlockSpec((1, D, B), lambda l: (l, 0, 0)),
            out_shape=jax.ShapeDtypeStruct((L, D, B), jnp.float32),
        )
    return pl.pallas_call(
        _transpose_block_acc,
        grid=(LS,),
        in_specs=[
            pl.BlockSpec((ROWS_PER_L, 128), lambda l: (l, 0)),
            pl.BlockSpec(memory_space=pl.ANY),
        ],
        out_specs=pl.BlockSpec((1, D, B), lambda l, _s=s: (_s * LS + l, 0, 0)),
        out_shape=jax.ShapeDtypeStruct((L, D, B), jnp.float32),
        input_output_aliases={1: 0},
    )


_to_ldb_slices = [_make_to_ldb(s) for s in range(S)]


def kernel(x, table):
    xt = jnp.transpose(x)
    ldb = None
    for s in range(S):
        g = _embed_slices[s](table, xt)
        flat = g.reshape(LS * ROWS_PER_L, 128)
        ldb = _to_ldb_slices[s](flat) if s == 0 else _to_ldb_slices[s](flat, ldb)
    return jnp.transpose(ldb, (0, 2, 1))


# speedup vs baseline: 1.0813x; 1.0008x over previous
import functools

import jax
import jax.numpy as jnp
from jax import lax
from jax.experimental import pallas as pl
from jax.experimental.pallas import tpu as pltpu
from jax.experimental.pallas import tpu_sc as plsc

L = 200
B = 4096
D = 32
CHUNK = 128
NC, NS = 2, 16
NW = NC * NS
NCHUNKS = (L * B) // CHUNK
CPL = B // CHUNK
NBUF = 5
PG = 2
KS = 2
S = 4
LS = L // S
NCH_S = NCHUNKS // S
ROWS_PER_L = (B * D) // 128

_mesh = plsc.VectorSubcoreMesh(
    core_axis_name="c", subcore_axis_name="s", num_cores=NC, num_subcores=NS
)


def _make_embed(nchunks, s0):
    cpw = nchunks // NW

    @functools.partial(
        pl.kernel,
        out_type=jax.ShapeDtypeStruct((nchunks, CHUNK, D), jnp.float32),
        mesh=_mesh,
        scratch_types=[
            pltpu.VMEM((3 * B,), jnp.int32),
            pltpu.VMEM((cpw, CHUNK), jnp.int32),
            pltpu.VMEM((NBUF, CHUNK, D), jnp.float32),
            pltpu.SemaphoreType.DMA,
            pltpu.SemaphoreType.DMA,
        ],
        compiler_params=pltpu.CompilerParams(
            use_tc_tiling_on_sc=False, needs_layout_passes=False
        ),
    )
    def embed(table_hbm, xt_hbm, out_hbm, xrow_v, idx_v, ring, sem_g, sem_s):
        wid = lax.axis_index("s") * NC + lax.axis_index("c")
        c0 = wid * cpw
        gc0 = s0 + c0

        lo = jnp.minimum(gc0 // CPL, L - 3)
        pltpu.sync_copy(xt_hbm.at[pl.ds(lo * B, 3 * B)], xrow_v)

        vi = lax.iota(jnp.int32, 16)
        v0 = (B // 4) * (vi & 3) + (vi >> 2)

        def perm_body(j, _):
            gc = gc0 + j
            roff = (gc // CPL - lo) * B + (gc % CPL) * (CHUNK // 4)
            for t in range(CHUNK // 16):
                vals = plsc.load_gather(xrow_v, [v0 + (roff + 4 * t)])
                idx_v[j, pl.ds(t * 16, 16)] = vals
            return 0

        lax.fori_loop(0, cpw, perm_body, 0)

        def issue_gather(j):
            pltpu.async_copy(
                table_hbm.at[idx_v.at[j]], ring.at[j % NBUF], sem_g
            )

        def wait_gather():
            pltpu.make_async_copy(
                table_hbm.at[idx_v.at[0]], ring.at[0], sem_g
            ).wait()

        def issue_scatter(j):
            pltpu.async_copy(ring.at[j % NBUF], out_hbm.at[c0 + j], sem_s)

        def wait_scatter():
            pltpu.make_async_copy(ring.at[0], out_hbm.at[0], sem_s).wait()

        for j in range(PG + 1):
            issue_gather(j)

        def body(j, _):
            wait_gather()
            issue_scatter(j)

            @pl.when(j >= KS)
            def _():
                wait_scatter()

            @pl.when(j + PG + 1 < cpw)
            def _():
                issue_gather(j + PG + 1)

            return 0

        lax.fori_loop(0, cpw, body, 0)
        for _ in range(KS):
            wait_scatter()

    return embed


_embed_slices = [_make_embed(NCH_S, s * NCH_S) for s in range(S)]


def _transpose_block(flat_ref, out_ref):
    xt = flat_ref[...].T
    for h in range(4):
        out_ref[0, :, h * ROWS_PER_L:(h + 1) * ROWS_PER_L] = (
            xt[h * D:(h + 1) * D, :]
        )


def _transpose_block_acc(flat_ref, acc_ref, out_ref):
    del acc_ref
    _transpose_block(flat_ref, out_ref)


def _make_to_ldb(s):
    if s == 0:
        return pl.pallas_call(
            _transpose_block,
            grid=(LS,),
            in_specs=[pl.BlockSpec((ROWS_PER_L, 128), lambda l: (l, 0))],
            out_specs=pl.BlockSpec((1, D, B), lambda l: (l, 0, 0)),
            out_shape=jax.ShapeDtypeStruct((L, D, B), jnp.float32),
        )
    return pl.pallas_call(
        _transpose_block_acc,
        grid=(LS,),
        in_specs=[
            pl.BlockSpec((ROWS_PER_L, 128), lambda l: (l, 0)),
            pl.BlockSpec(memory_space=pl.ANY),
        ],
        out_specs=pl.BlockSpec((1, D, B), lambda l, _s=s: (_s * LS + l, 0, 0)),
        out_shape=jax.ShapeDtypeStruct((L, D, B), jnp.float32),
        input_output_aliases={1: 0},
    )


_to_ldb_slices = [_make_to_ldb(s) for s in range(S)]


def kernel(x, table):
    xt = jnp.transpose(x).reshape(L * B)
    ldb = None
    for s in range(S):
        g = _embed_slices[s](table, xt)
        flat = g.reshape(LS * ROWS_PER_L, 128)
        ldb = _to_ldb_slices[s](flat) if s == 0 else _to_ldb_slices[s](flat, ldb)
    return jnp.transpose(ldb, (0, 2, 1))
